# Initial kernel scaffold; baseline (speedup 1.0000x reference)
import jax
import jax.numpy as jnp
from jax.experimental import pallas as pl


def kernel(x, y):
    raise NotImplementedError("write your pallas kernel here")


# speedup vs baseline: 2.0156x; 2.0156x over previous
import functools

import jax
import jax.numpy as jnp
from jax import lax
from jax.experimental import pallas as pl
from jax.experimental.pallas import tpu as pltpu
from jax.experimental.pallas import tpu_sc as plsc

R = 64
N = 32768
L = 16
SEG = N // L
BITS = 8
BINS = 1 << BITS
NC = 2
NS = 16
NW = NC * NS
ROWS_PER_W = R // NW

def _fwd_mono(v_f32):
    i = lax.bitcast_convert_type(v_f32, jnp.int32)
    flip = lax.bitwise_or(lax.shift_right_arithmetic(i, 31),
                          jnp.int32(-2147483648))
    return lax.bitwise_xor(i, flip)


def _inv_mono(m_i32):
    flip = lax.bitwise_or(
        lax.bitwise_not(lax.shift_right_arithmetic(m_i32, 31)),
        jnp.int32(-2147483648))
    return lax.bitcast_convert_type(lax.bitwise_xor(m_i32, flip), jnp.float32)


def _body(x_hbm, y_hbm, out_hbm, bufx, bufy, bufs, hist, accv):
    wid = lax.axis_index("s") * NC + lax.axis_index("c")
    lane = lax.iota(jnp.int32, L)
    seg_base = lane * SEG
    ones_i = jnp.ones((L,), jnp.int32)
    zeros_i = jnp.zeros((L,), jnp.int32)

    def transform(buf):
        def tf(t, _):
            sl = pl.ds(t * L, L)
            buf[sl] = lax.bitcast_convert_type(_fwd_mono(buf[sl]), jnp.float32)
            return 0
        lax.fori_loop(0, SEG, tf, 0)

    def radix_pass(src, dst, shift):
        def z(i, _):
            hist[pl.ds(i * L, L)] = zeros_i
            return 0
        lax.fori_loop(0, BINS, z, 0)

        def h(t, _):
            v = plsc.load_gather(src, [seg_base + t])
            m = lax.bitcast_convert_type(v, jnp.int32)
            d = lax.bitwise_and(
                lax.shift_right_logical(m, shift), jnp.int32(BINS - 1))
            plsc.addupdate_scatter(hist, [lax.shift_left(d, 4) + lane], ones_i)
            return 0
        lax.fori_loop(0, SEG, h, 0)

        def s(i, carry):
            sl = pl.ds(i * L, L)
            hv = hist[sl]
            hist[sl] = jnp.cumsum(hv) - hv + carry
            return carry + jnp.sum(hv)
        lax.fori_loop(0, BINS, s, jnp.int32(0))

        def p(t, _):
            v = plsc.load_gather(src, [seg_base + t])
            m = lax.bitcast_convert_type(v, jnp.int32)
            d = lax.bitwise_and(
                lax.shift_right_logical(m, shift), jnp.int32(BINS - 1))
            hidx = lax.shift_left(d, 4) + lane
            o = plsc.load_gather(hist, [hidx])
            plsc.store_scatter(dst, [o], v)
            plsc.store_scatter(hist, [hidx], o + ones_i)
            return 0
        lax.fori_loop(0, SEG, p, 0)

    def sort_inplace(buf):
        radix_pass(buf, bufs, 0)
        radix_pass(bufs, buf, 8)
        radix_pass(buf, bufs, 16)
        radix_pass(bufs, buf, 24)

    acc = jnp.zeros((L,), jnp.float32)
    for r in range(ROWS_PER_W):
        row = wid * ROWS_PER_W + r
        pltpu.sync_copy(x_hbm.at[row], bufx)
        pltpu.sync_copy(y_hbm.at[row], bufy)
        transform(bufx)
        transform(bufy)
        sort_inplace(bufx)
        sort_inplace(bufy)

        def dacc(t, a):
            sl = pl.ds(t * L, L)
            fx = _inv_mono(lax.bitcast_convert_type(bufx[sl], jnp.int32))
            fy = _inv_mono(lax.bitcast_convert_type(bufy[sl], jnp.int32))
            diff = fx - fy
            return a + diff * diff
        acc = lax.fori_loop(0, SEG, dacc, acc)

    accv[...] = acc * jnp.float32(1.0 / N)
    pltpu.sync_copy(accv, out_hbm.at[wid])


@jax.jit
def _qag_partials(x, y):
    mesh = plsc.VectorSubcoreMesh(core_axis_name="c", subcore_axis_name="s")
    f = pl.kernel(
        _body,
        out_type=jax.ShapeDtypeStruct((NW, L), jnp.float32),
        mesh=mesh,
        compiler_params=pltpu.CompilerParams(
            needs_layout_passes=False,
            use_tc_tiling_on_sc=False,
        ),
        scratch_types=[
            pltpu.VMEM((N,), jnp.float32),
            pltpu.VMEM((N,), jnp.float32),
            pltpu.VMEM((N,), jnp.float32),
            pltpu.VMEM((BINS * L,), jnp.int32),
            pltpu.VMEM((L,), jnp.float32),
        ],
    )
    return f(x, y)


def kernel(x, y):
    return jnp.sum(_qag_partials(x, y))
